# baseline (speedup 1.0000x reference)
import functools

import jax
import jax.numpy as jnp
from jax import lax
from jax.experimental import pallas as pl
from jax.experimental.pallas import tpu as pltpu
from jax.experimental.pallas import tpu_sc as plsc

BATCH = 16384
FIELDS = 26
EMBEDDING_DIM = 64

NUM_CORES = 2
NUM_SUBCORES = 16
NW = NUM_CORES * NUM_SUBCORES

B_TOTAL = BATCH * FIELDS
BLK = 128
N_BLOCKS = B_TOTAL // BLK
BLOCKS_PER_W = N_BLOCKS // NW
ROWS_PER_W = BLOCKS_PER_W * BLK

_mesh = plsc.VectorSubcoreMesh(
    core_axis_name="c", subcore_axis_name="s",
    num_cores=NUM_CORES, num_subcores=NUM_SUBCORES)


@functools.partial(
    pl.kernel,
    out_type=jax.ShapeDtypeStruct((B_TOTAL, EMBEDDING_DIM), jnp.float32),
    mesh=_mesh,
    scratch_types=[
        pltpu.VMEM((BLOCKS_PER_W, BLK), jnp.int32),
        pltpu.VMEM((BLK, EMBEDDING_DIM), jnp.float32),
        pltpu.SemaphoreType.DMA,
    ],
    compiler_params=pltpu.CompilerParams(use_tc_tiling_on_sc=False),
)
def _gather_kernel(idx_hbm, table_hbm, out_hbm, idx_v, rows_v, sem):
    wid = lax.axis_index("s") * NUM_CORES + lax.axis_index("c")
    blk_base = wid * BLOCKS_PER_W
    row_base = wid * ROWS_PER_W
    pltpu.sync_copy(idx_hbm.at[pl.ds(blk_base, BLOCKS_PER_W)], idx_v)

    @pl.loop(0, BLOCKS_PER_W)
    def _body(g):
        pltpu.async_copy(table_hbm.at[idx_v.at[g]], rows_v, sem).wait()
        pltpu.sync_copy(rows_v, out_hbm.at[pl.ds(row_base + g * BLK, BLK)])


def kernel(x, weight):
    idx = x.reshape(N_BLOCKS, BLK).astype(jnp.int32)
    out = _gather_kernel(idx, weight)
    return out.reshape(BATCH, FIELDS, EMBEDDING_DIM)


# speedup vs baseline: 1.0779x; 1.0779x over previous
import functools

import jax
import jax.numpy as jnp
from jax import lax
from jax.experimental import pallas as pl
from jax.experimental.pallas import tpu as pltpu
from jax.experimental.pallas import tpu_sc as plsc

BATCH = 16384
FIELDS = 26
EMBEDDING_DIM = 64

NUM_CORES = 2
NUM_SUBCORES = 16
NW = NUM_CORES * NUM_SUBCORES

B_TOTAL = BATCH * FIELDS
BLK = 128
N_BLOCKS = B_TOTAL // BLK
BLOCKS_PER_W = N_BLOCKS // NW
NBUF = 4
ROWS_PER_W = BLOCKS_PER_W * BLK

_mesh = plsc.VectorSubcoreMesh(
    core_axis_name="c", subcore_axis_name="s",
    num_cores=NUM_CORES, num_subcores=NUM_SUBCORES)


@functools.partial(
    pl.kernel,
    out_type=jax.ShapeDtypeStruct((B_TOTAL, EMBEDDING_DIM), jnp.float32),
    mesh=_mesh,
    scratch_types=[
        pltpu.VMEM((BLOCKS_PER_W, BLK), jnp.int32),
        pltpu.VMEM((NBUF, BLK, EMBEDDING_DIM), jnp.float32),
        [pltpu.SemaphoreType.DMA] * NBUF,
    ],
    compiler_params=pltpu.CompilerParams(use_tc_tiling_on_sc=False),
)
def _gather_kernel(idx_hbm, table_hbm, out_hbm, idx_v, rows_v, sems):
    wid = lax.axis_index("s") * NUM_CORES + lax.axis_index("c")
    blk_base = wid * BLOCKS_PER_W
    row_base = wid * ROWS_PER_W
    pltpu.sync_copy(idx_hbm.at[pl.ds(blk_base, BLOCKS_PER_W)], idx_v)

    def start_gather(g, b):
        pltpu.async_copy(table_hbm.at[idx_v.at[g]], rows_v.at[b], sems[b])

    def wait_gather(b):
        pltpu.make_async_copy(table_hbm.at[idx_v.at[0]], rows_v.at[b],
                              sems[b]).wait()

    for b in range(NBUF - 1):
        start_gather(b, b)

    @pl.loop(0, BLOCKS_PER_W // NBUF)
    def _body(j):
        for b in range(NBUF):
            g = j * NBUF + b
            wait_gather(b)
            gnext = g + NBUF - 1

            @pl.when(gnext < BLOCKS_PER_W)
            def _():
                start_gather(gnext, (b + NBUF - 1) % NBUF)

            pltpu.sync_copy(rows_v.at[b],
                            out_hbm.at[pl.ds(row_base + g * BLK, BLK)])


def kernel(x, weight):
    idx = x.reshape(N_BLOCKS, BLK).astype(jnp.int32)
    out = _gather_kernel(idx, weight)
    return out.reshape(BATCH, FIELDS, EMBEDDING_DIM)
